# baseline (speedup 1.0000x reference)
import functools
import math

import jax
import jax.numpy as jnp
from jax import lax
from jax.experimental import pallas as pl
from jax.experimental.pallas import tpu as pltpu
from jax.experimental.pallas import tpu_sc as plsc

NUM_EMB = 100000
INNER = 16
EMB = 96
B = 16384
L = 200


_ROWS_PER_BLOCK = 4000
EMB_PAD = 128


def _transform_body(table_ref, w_ref, b_ref, gamma_ref, beta_ref, out_ref):
    x = jnp.dot(table_ref[...], w_ref[...], preferred_element_type=jnp.float32)
    x = x + b_ref[...]
    mean = jnp.sum(x, axis=-1, keepdims=True) * (1.0 / EMB)
    mask = lax.broadcasted_iota(jnp.int32, x.shape, 1) < EMB
    xc = jnp.where(mask, x - mean, 0.0)
    var = jnp.sum(xc * xc, axis=-1, keepdims=True) * (1.0 / EMB)
    y = xc * lax.rsqrt(var + 1e-5)
    y = y * gamma_ref[...] + beta_ref[...]
    out_ref[...] = y * 0.5 * (1.0 + lax.erf(y * (1.0 / math.sqrt(2.0))))


def _transform_table(table, W, b2, gamma2, beta2):
    grid = (NUM_EMB // _ROWS_PER_BLOCK,)
    return pl.pallas_call(
        _transform_body,
        grid=grid,
        in_specs=[
            pl.BlockSpec((_ROWS_PER_BLOCK, INNER), lambda i: (i, 0)),
            pl.BlockSpec((INNER, EMB_PAD), lambda i: (0, 0)),
            pl.BlockSpec((1, EMB_PAD), lambda i: (0, 0)),
            pl.BlockSpec((1, EMB_PAD), lambda i: (0, 0)),
            pl.BlockSpec((1, EMB_PAD), lambda i: (0, 0)),
        ],
        out_specs=pl.BlockSpec((_ROWS_PER_BLOCK, EMB_PAD), lambda i: (i, 0)),
        out_shape=jax.ShapeDtypeStruct((NUM_EMB, EMB_PAD), jnp.float32),
    )(table, W, b2, gamma2, beta2)


N = B * L
_NC, _NS = 2, 16
_NW = _NC * _NS
_CHUNK = 400


@functools.cache
def _make_gather_kernel(ntok):
    per_w = ntok // _NW
    n_chunks = per_w // _CHUNK

    @functools.partial(
        pl.kernel,
        mesh=plsc.VectorSubcoreMesh(core_axis_name="c", subcore_axis_name="s"),
        out_type=jax.ShapeDtypeStruct((ntok, EMB_PAD), jnp.float32),
        scratch_types=[
            pltpu.VMEM((_CHUNK,), jnp.int32),
            pltpu.VMEM((_CHUNK,), jnp.int32),
            pltpu.VMEM((_CHUNK, EMB_PAD), jnp.float32),
            pltpu.VMEM((_CHUNK, EMB_PAD), jnp.float32),
            pltpu.SemaphoreType.DMA,
            pltpu.SemaphoreType.DMA,
            pltpu.SemaphoreType.DMA,
            pltpu.SemaphoreType.DMA,
        ],
    )
    def _gather_kernel(table_hbm, idx_hbm, out_hbm,
                       idxa, idxb, raw0, raw1,
                       g0, g1, o0, o1):
        wid = lax.axis_index("s") * _NC + lax.axis_index("c")
        tbase = wid * per_w
        raw = (raw0, raw1)
        gsem = (g0, g1)
        osem = (o0, o1)
        idx_v = (idxa, idxb)

        def issue_gather(s, k):
            pltpu.sync_copy(idx_hbm.at[pl.ds(tbase + k * _CHUNK, _CHUNK)],
                            idx_v[s])
            pltpu.async_copy(table_hbm.at[idx_v[s]], raw[s], gsem[s])

        def wait_gather(s, k):
            pltpu.make_async_copy(table_hbm.at[idx_v[s]], raw[s],
                                  gsem[s]).wait()

        def out_dst(k):
            return out_hbm.at[pl.ds(tbase + k * _CHUNK, _CHUNK)]

        issue_gather(0, 0)
        issue_gather(1, 1)

        def handle(s, k):
            wait_gather(s, k)
            pltpu.async_copy(raw[s], out_dst(k), osem[s])

            @pl.when(k + 2 < n_chunks)
            def _():
                pltpu.make_async_copy(raw[s], out_dst(k), osem[s]).wait()
                issue_gather(s, k + 2)

        def body(i2, carry):
            handle(0, 2 * i2)
            handle(1, 2 * i2 + 1)
            return carry

        lax.fori_loop(0, n_chunks // 2, body, 0)

        pltpu.make_async_copy(raw[0], out_dst(n_chunks - 2),
                              osem[0]).wait()
        pltpu.make_async_copy(raw[1], out_dst(n_chunks - 1),
                              osem[1]).wait()

    return _gather_kernel


_BB = 16
_G1 = (N // 2) // (_BB * L)


def _merge_body(h1_ref, h2_ref, out_ref):
    i = pl.program_id(0)

    @pl.when(i < _G1)
    def _():
        out_ref[...] = h1_ref[:, :EMB].reshape(_BB, L, EMB)

    @pl.when(i >= _G1)
    def _():
        out_ref[...] = h2_ref[:, :EMB].reshape(_BB, L, EMB)


def _merge(h1, h2):
    return pl.pallas_call(
        _merge_body,
        grid=(2 * _G1,),
        in_specs=[
            pl.BlockSpec((_BB * L, EMB_PAD),
                         lambda i: (jnp.minimum(i, _G1 - 1), 0)),
            pl.BlockSpec((_BB * L, EMB_PAD),
                         lambda i: (jnp.maximum(i - _G1, 0), 0)),
        ],
        out_specs=pl.BlockSpec((_BB, L, EMB), lambda i: (i, 0, 0)),
        out_shape=jax.ShapeDtypeStruct((B, L, EMB), jnp.float32),
    )(h1, h2)


def kernel(embryo_type, table, W, b, gamma, beta):
    pad = EMB_PAD - EMB
    table2 = _transform_table(
        table,
        jnp.pad(W, ((0, 0), (0, pad))),
        jnp.pad(b.reshape(1, EMB), ((0, 0), (0, pad))),
        jnp.pad(gamma.reshape(1, EMB), ((0, 0), (0, pad))),
        jnp.pad(beta.reshape(1, EMB), ((0, 0), (0, pad))),
    )
    idx = embryo_type.reshape(N).astype(jnp.int32)
    gk = _make_gather_kernel(N // 2)
    h1 = gk(table2, idx[:N // 2])
    h2 = gk(table2, idx[N // 2:])
    return _merge(h1, h2)
